# baseline (speedup 1.0000x reference)
import functools

import jax
import jax.numpy as jnp
from jax import lax
from jax.experimental import pallas as pl
from jax.experimental.pallas import tpu as pltpu
from jax.experimental.pallas import tpu_sc as plsc

N = 10000
E = 320000
IN_DIM = 128
OUT_DIM = 128
EH = 32
NC = 2
NS = 16
NW = NC * NS
EPW = E // NW
CH = 40
GPC = 25
GE = CH * GPC
NG = EPW // GE
RPW = EPW // CH
SCH = 80
SGPC = 25
SGE = SCH * SGPC
SNG = EPW // SGE
SRPW = EPW // SCH
RPS = N // NS


def _sig(t):
    return 1.0 / (1.0 + jnp.exp(-t))


def _proj_body(x_ref, wa_ref, wb_ref, xa_ref, xb_ref):
    xv = x_ref[...]
    xa_ref[...] = jnp.dot(xv, wa_ref[...],
                          preferred_element_type=jnp.float32).astype(jnp.bfloat16)
    xb_ref[...] = jnp.dot(xv, wb_ref[...],
                          preferred_element_type=jnp.float32).astype(jnp.bfloat16)


def _proj(x2, w1a, w1b):
    bn = 2000
    return pl.pallas_call(
        _proj_body,
        grid=(N // bn,),
        in_specs=[
            pl.BlockSpec((bn, IN_DIM), lambda i: (i, 0)),
            pl.BlockSpec((IN_DIM, EH), lambda i: (0, 0)),
            pl.BlockSpec((IN_DIM, EH), lambda i: (0, 0)),
        ],
        out_specs=[
            pl.BlockSpec((bn, EH), lambda i: (i, 0)),
            pl.BlockSpec((bn, EH), lambda i: (i, 0)),
        ],
        out_shape=[
            jax.ShapeDtypeStruct((N, EH), jnp.bfloat16),
            jax.ShapeDtypeStruct((N, EH), jnp.bfloat16),
        ],
    )(x2, w1a, w1b)


def _ewc_body(ew0_ref, ew1_ref, ew2_ref, ew3_ref, w1c_ref, o_ref):
    dn = (((0,), (0,)), ((), ()))
    w1c = w1c_ref[...]
    o_ref[...] = jnp.concatenate(
        [lax.dot_general(r[...], w1c, dn, preferred_element_type=jnp.float32)
         for r in (ew0_ref, ew1_ref, ew2_ref, ew3_ref)],
        axis=1).astype(jnp.bfloat16)


def _ewc(ewT, w1c):
    br = 3200
    ep = E // 4
    nb = ep // br
    return pl.pallas_call(
        _ewc_body,
        grid=(nb,),
        in_specs=[
            pl.BlockSpec((3, br), lambda i: (0, i)),
            pl.BlockSpec((3, br), lambda i: (0, i + nb)),
            pl.BlockSpec((3, br), lambda i: (0, i + 2 * nb)),
            pl.BlockSpec((3, br), lambda i: (0, i + 3 * nb)),
            pl.BlockSpec((3, EH), lambda i: (0, 0)),
        ],
        out_specs=pl.BlockSpec((br, 128), lambda i: (i, 0)),
        out_shape=jax.ShapeDtypeStruct((ep, 128), jnp.bfloat16),
    )(ewT, ewT, ewT, ewT, w1c)


def _gather_body(xa_hbm, xb_hbm, src_hbm, tgt_hbm, ga_hbm, gb_hbm,
                 src_i, tgt_i, ga_v, gb_v, sem):
    c = lax.axis_index("c")
    s = lax.axis_index("s")
    wid = c * NS + s

    qw = wid // (NW // 4)
    lr0 = (wid % (NW // 4)) * EPW

    def group(g, carry):
        row0 = wid * RPW + g * GPC
        rr = lr0 + g * GE
        pltpu.sync_copy(src_hbm.at[pl.ds(row0, GPC)], src_i)
        pltpu.sync_copy(tgt_hbm.at[pl.ds(row0, GPC)], tgt_i)

        def burst(j, c2):
            for u in range(5):
                k = j * 5 + u
                pltpu.async_copy(xa_hbm.at[src_i.at[k]],
                                 ga_v.at[pl.ds(k * CH, CH)], sem)
                pltpu.async_copy(xb_hbm.at[tgt_i.at[k]],
                                 gb_v.at[pl.ds(k * CH, CH)], sem)
            pltpu.make_async_copy(xa_hbm.at[pl.ds(0, 5 * CH)],
                                  ga_v.at[pl.ds(j * 5 * CH, 5 * CH)],
                                  sem).wait()
            pltpu.make_async_copy(xb_hbm.at[pl.ds(0, 5 * CH)],
                                  gb_v.at[pl.ds(j * 5 * CH, 5 * CH)],
                                  sem).wait()
            return c2

        lax.fori_loop(0, GPC // 5, burst, 0)
        pltpu.sync_copy(ga_v, ga_hbm.at[pl.ds(rr, GE), pl.ds(EH * qw, EH)])
        pltpu.sync_copy(gb_v, gb_hbm.at[pl.ds(rr, GE), pl.ds(EH * qw, EH)])
        return carry

    lax.fori_loop(0, NG, group, 0)


def _gather(xa, xb, src2, tgt2):
    mesh = plsc.VectorSubcoreMesh(core_axis_name="c", subcore_axis_name="s")
    f = pl.kernel(
        _gather_body,
        out_type=[
            jax.ShapeDtypeStruct((E // 4, 128), jnp.bfloat16),
            jax.ShapeDtypeStruct((E // 4, 128), jnp.bfloat16),
        ],
        mesh=mesh,
        compiler_params=pltpu.CompilerParams(use_tc_tiling_on_sc=False),
        scratch_types=[
            pltpu.VMEM((GPC, CH), jnp.int32),
            pltpu.VMEM((GPC, CH), jnp.int32),
            pltpu.VMEM((GE, EH), jnp.bfloat16),
            pltpu.VMEM((GE, EH), jnp.bfloat16),
            pltpu.SemaphoreType.DMA,
        ],
    )
    return f(xa, xb, src2, tgt2)


def _edge_body(ga_ref, gb_ref, ewc_ref, b1_ref, w2_ref, b2_ref, o_ref):
    pre1 = (ga_ref[...].astype(jnp.float32) + gb_ref[...].astype(jnp.float32)
            + ewc_ref[...].astype(jnp.float32) + b1_ref[...])
    h1 = _sig(pre1)
    o_ref[...] = _sig(jnp.dot(h1, w2_ref[...],
                              preferred_element_type=jnp.float32)
                      + b2_ref[...])


def _edge_mlp(ga_p, gb_p, ewc_p, b1t, w2_blk, b2t):
    br = 3200
    ep = E // 4
    nb = ep // br
    return pl.pallas_call(
        _edge_body,
        grid=(nb,),
        in_specs=[
            pl.BlockSpec((br, 128), lambda i: (i, 0)),
            pl.BlockSpec((br, 128), lambda i: (i, 0)),
            pl.BlockSpec((br, 128), lambda i: (i, 0)),
            pl.BlockSpec((1, 128), lambda i: (0, 0)),
            pl.BlockSpec((128, 128), lambda i: (0, 0)),
            pl.BlockSpec((1, 128), lambda i: (0, 0)),
        ],
        out_specs=pl.BlockSpec((br, 128), lambda i: (i, 0)),
        out_shape=jax.ShapeDtypeStruct((ep, 128), jnp.float32),
    )(ga_p, gb_p, ewc_p, b1t, w2_blk, b2t)


def _scatter_body(h2_hbm, src_hbm, tgt_hbm, zer_hbm, out_hbm,
                  src_i, tgt_i, h2_v, acc_a, acc_b, sem):
    c = lax.axis_index("c")
    s = lax.axis_index("s")
    wid = c * NS + s
    r0 = s * RPS
    qw = wid // (NW // 4)
    lr0 = (wid % (NW // 4)) * EPW

    pltpu.sync_copy(zer_hbm, acc_a.at[pl.ds(r0, RPS)])
    pltpu.sync_copy(zer_hbm, acc_b.at[pl.ds(r0, RPS)])
    plsc.subcore_barrier()

    def group(g, carry):
        row0 = wid * SRPW + g * SGPC
        rr = lr0 + g * SGE
        pltpu.sync_copy(h2_hbm.at[pl.ds(rr, SGE), pl.ds(EH * qw, EH)], h2_v)
        pltpu.sync_copy(src_hbm.at[pl.ds(row0, SGPC)], src_i)
        pltpu.sync_copy(tgt_hbm.at[pl.ds(row0, SGPC)], tgt_i)

        def burst(j, c2):
            for u in range(5):
                k = j * 5 + u
                sl = h2_v.at[pl.ds(k * SCH, SCH)]
                pltpu.sync_copy(sl, acc_a.at[tgt_i.at[k]], add=True)
                pltpu.sync_copy(sl, acc_b.at[src_i.at[k]], add=True)
            return c2

        lax.fori_loop(0, SGPC // 5, burst, 0)
        return carry

    lax.fori_loop(0, SNG, group, 0)
    plsc.subcore_barrier()

    pltpu.sync_copy(acc_a.at[pl.ds(r0, RPS)],
                    out_hbm.at[2 * c, pl.ds(r0, RPS)])
    pltpu.sync_copy(acc_b.at[pl.ds(r0, RPS)],
                    out_hbm.at[2 * c + 1, pl.ds(r0, RPS)])


def _scatter(h2, src2, tgt2, zer):
    mesh = plsc.VectorSubcoreMesh(core_axis_name="c", subcore_axis_name="s")
    f = pl.kernel(
        _scatter_body,
        out_type=jax.ShapeDtypeStruct((4, N, EH), jnp.float32),
        name="scatter_sc",
        mesh=mesh,
        compiler_params=pltpu.CompilerParams(use_tc_tiling_on_sc=False),
        scratch_types=[
            pltpu.VMEM((SGPC, SCH), jnp.int32),
            pltpu.VMEM((SGPC, SCH), jnp.int32),
            pltpu.VMEM((SGE, EH), jnp.float32),
            pltpu.VMEM_SHARED((N, EH), jnp.float32),
            pltpu.VMEM_SHARED((N, EH), jnp.float32),
            pltpu.SemaphoreType.DMA,
        ],
    )
    return f(h2, src2, tgt2, zer)


def _node_body(p_ref, w3_ref, b3_ref, o_ref):
    p = p_ref[...]
    agg = (p[0] + p[2]) - (p[1] + p[3])
    o_ref[...] = _sig(jnp.dot(agg, w3_ref[...],
                              preferred_element_type=jnp.float32)
                      + b3_ref[...])


def _node_mlp(parts, w3p, b3r):
    bn = 2000
    return pl.pallas_call(
        _node_body,
        grid=(N // bn,),
        in_specs=[
            pl.BlockSpec((4, bn, EH), lambda i: (0, i, 0)),
            pl.BlockSpec((EH, OUT_DIM), lambda i: (0, 0)),
            pl.BlockSpec((1, OUT_DIM), lambda i: (0, 0)),
        ],
        out_specs=pl.BlockSpec((bn, OUT_DIM), lambda i: (i, 0)),
        out_shape=jax.ShapeDtypeStruct((N, OUT_DIM), jnp.float32),
    )(parts, w3p, b3r)


def kernel(x, edge_w, edge_index, W1, b1, W2, b2, W3, b3):
    x2 = x[0].astype(jnp.float32)
    ewT = edge_w[0].T.astype(jnp.float32)
    src = edge_index[0].astype(jnp.int32)
    tgt = edge_index[1].astype(jnp.int32)

    w1a = W1[:IN_DIM]
    w1b = W1[IN_DIM:2 * IN_DIM]
    w1c = W1[2 * IN_DIM:]
    eye4 = jnp.eye(4, dtype=jnp.float32)
    b1t = jnp.tile(b1, 4).reshape(1, 128)
    w2p = jnp.zeros((EH, EH), jnp.float32).at[:, :W2.shape[1]].set(W2)
    b2p = jnp.zeros((EH,), jnp.float32).at[:b2.shape[0]].set(b2)
    w2_blk = jnp.kron(eye4, w2p)
    b2t = jnp.tile(b2p, 4).reshape(1, 128)
    w3p = jnp.zeros((EH, OUT_DIM), jnp.float32).at[:W3.shape[0]].set(W3)
    b3r = b3.reshape(1, OUT_DIM)
    zer = jnp.zeros((RPS, EH), jnp.float32)
    src2 = src.reshape(E // CH, CH)
    tgt2 = tgt.reshape(E // CH, CH)
    src3 = src.reshape(E // SCH, SCH)
    tgt3 = tgt.reshape(E // SCH, SCH)

    xa, xb = _proj(x2, w1a, w1b)
    ewc_p = _ewc(ewT, w1c)
    ga, gb = _gather(xa, xb, src2, tgt2)
    h2_p = _edge_mlp(ga, gb, ewc_p, b1t, w2_blk, b2t)
    parts = _scatter(h2_p, src3, tgt3, zer)
    y = _node_mlp(parts, w3p, b3r)
    return y[None]


# speedup vs baseline: 1.6185x; 1.6185x over previous
import functools

import jax
import jax.numpy as jnp
from jax import lax
from jax.experimental import pallas as pl
from jax.experimental.pallas import tpu as pltpu
from jax.experimental.pallas import tpu_sc as plsc

N = 10000
E = 320000
IN_DIM = 128
OUT_DIM = 128
EH = 32
NC = 2
NS = 16
NW = NC * NS
EPW = E // NW
CH = 40
GPC = 25
GE = CH * GPC
NG = EPW // GE
RPW = EPW // CH
SCH = 80
SGPC = 25
SGE = SCH * SGPC
SNG = EPW // SGE
SRPW = EPW // SCH
RPS = N // NS


def _sig(t):
    return 1.0 / (1.0 + jnp.exp(-t))


def _proj_body(x_ref, wa_ref, wb_ref, xa_ref, xb_ref):
    xv = x_ref[...]
    xa_ref[...] = jnp.dot(xv, wa_ref[...],
                          preferred_element_type=jnp.float32)
    xb_ref[...] = jnp.dot(xv, wb_ref[...],
                          preferred_element_type=jnp.float32)


def _proj(x2, w1a, w1b):
    bn = 2000
    return pl.pallas_call(
        _proj_body,
        grid=(N // bn,),
        in_specs=[
            pl.BlockSpec((bn, IN_DIM), lambda i: (i, 0)),
            pl.BlockSpec((IN_DIM, EH), lambda i: (0, 0)),
            pl.BlockSpec((IN_DIM, EH), lambda i: (0, 0)),
        ],
        out_specs=[
            pl.BlockSpec((bn, EH), lambda i: (i, 0)),
            pl.BlockSpec((bn, EH), lambda i: (i, 0)),
        ],
        out_shape=[
            jax.ShapeDtypeStruct((N, EH), jnp.float32),
            jax.ShapeDtypeStruct((N, EH), jnp.float32),
        ],
    )(x2, w1a, w1b)


def _ewc_body(ew0_ref, ew1_ref, ew2_ref, ew3_ref, w1c_ref, o_ref):
    dn = (((0,), (0,)), ((), ()))
    w1c = w1c_ref[...]
    o_ref[...] = jnp.concatenate(
        [lax.dot_general(r[...], w1c, dn, preferred_element_type=jnp.float32)
         for r in (ew0_ref, ew1_ref, ew2_ref, ew3_ref)],
        axis=1)


def _ewc(ewT, w1c):
    br = 3200
    ep = E // 4
    nb = ep // br
    return pl.pallas_call(
        _ewc_body,
        grid=(nb,),
        in_specs=[
            pl.BlockSpec((3, br), lambda i: (0, i)),
            pl.BlockSpec((3, br), lambda i: (0, i + nb)),
            pl.BlockSpec((3, br), lambda i: (0, i + 2 * nb)),
            pl.BlockSpec((3, br), lambda i: (0, i + 3 * nb)),
            pl.BlockSpec((3, EH), lambda i: (0, 0)),
        ],
        out_specs=pl.BlockSpec((br, 128), lambda i: (i, 0)),
        out_shape=jax.ShapeDtypeStruct((ep, 128), jnp.float32),
    )(ewT, ewT, ewT, ewT, w1c)


def _gather_body(xa_hbm, xb_hbm, src_hbm, tgt_hbm, ga_hbm, gb_hbm,
                 src_i, tgt_i, ga_v, gb_v, sem, wsem):
    c = lax.axis_index("c")
    s = lax.axis_index("s")
    wid = c * NS + s

    qw = wid // (NW // 4)
    lr0 = (wid % (NW // 4)) * EPW
    sb = GPC // 5

    def drain_sub(buf_v, j):
        pltpu.make_async_copy(xa_hbm.at[pl.ds(0, 5 * CH)],
                              buf_v.at[pl.ds(j * 5 * CH, 5 * CH)],
                              sem).wait()

    def drain_writes():
        pltpu.make_async_copy(xa_hbm.at[pl.ds(0, GE)], ga_v, wsem).wait()
        pltpu.make_async_copy(xa_hbm.at[pl.ds(0, GE)], gb_v, wsem).wait()

    def group(g, carry):
        row0 = wid * RPW + g * GPC
        rr = lr0 + g * GE
        @pl.when(g > 0)
        def _():
            drain_writes()
        pltpu.sync_copy(src_hbm.at[pl.ds(row0, GPC)], src_i)
        pltpu.sync_copy(tgt_hbm.at[pl.ds(row0, GPC)], tgt_i)

        def burst(j, c2):
            for u in range(5):
                k = j * 5 + u
                pltpu.async_copy(xa_hbm.at[src_i.at[k]],
                                 ga_v.at[pl.ds(k * CH, CH)], sem)
                pltpu.async_copy(xb_hbm.at[tgt_i.at[k]],
                                 gb_v.at[pl.ds(k * CH, CH)], sem)
            @pl.when(j > 0)
            def _():
                drain_sub(ga_v, j - 1)
                drain_sub(gb_v, j - 1)
            return c2

        lax.fori_loop(0, sb, burst, 0)
        drain_sub(ga_v, sb - 1)
        drain_sub(gb_v, sb - 1)
        pltpu.async_copy(ga_v, ga_hbm.at[pl.ds(rr, GE), pl.ds(EH * qw, EH)],
                         wsem)
        pltpu.async_copy(gb_v, gb_hbm.at[pl.ds(rr, GE), pl.ds(EH * qw, EH)],
                         wsem)
        return carry

    lax.fori_loop(0, NG, group, 0)
    drain_writes()


def _gather(xa, xb, src2, tgt2):
    mesh = plsc.VectorSubcoreMesh(core_axis_name="c", subcore_axis_name="s")
    f = pl.kernel(
        _gather_body,
        out_type=[
            jax.ShapeDtypeStruct((E // 4, 128), jnp.float32),
            jax.ShapeDtypeStruct((E // 4, 128), jnp.float32),
        ],
        mesh=mesh,
        compiler_params=pltpu.CompilerParams(use_tc_tiling_on_sc=False),
        scratch_types=[
            pltpu.VMEM((GPC, CH), jnp.int32),
            pltpu.VMEM((GPC, CH), jnp.int32),
            pltpu.VMEM((GE, EH), jnp.float32),
            pltpu.VMEM((GE, EH), jnp.float32),
            pltpu.SemaphoreType.DMA,
            pltpu.SemaphoreType.DMA,
        ],
    )
    return f(xa, xb, src2, tgt2)


def _edge_body(ga_ref, gb_ref, ewc_ref, b1_ref, w2_ref, b2_ref, o_ref):
    pre1 = ga_ref[...] + gb_ref[...] + ewc_ref[...] + b1_ref[...]
    h1 = _sig(pre1)
    o_ref[...] = _sig(jnp.dot(h1, w2_ref[...],
                              preferred_element_type=jnp.float32)
                      + b2_ref[...])


def _edge_mlp(ga_p, gb_p, ewc_p, b1t, w2_blk, b2t):
    br = 3200
    ep = E // 4
    nb = ep // br
    return pl.pallas_call(
        _edge_body,
        grid=(nb,),
        in_specs=[
            pl.BlockSpec((br, 128), lambda i: (i, 0)),
            pl.BlockSpec((br, 128), lambda i: (i, 0)),
            pl.BlockSpec((br, 128), lambda i: (i, 0)),
            pl.BlockSpec((1, 128), lambda i: (0, 0)),
            pl.BlockSpec((128, 128), lambda i: (0, 0)),
            pl.BlockSpec((1, 128), lambda i: (0, 0)),
        ],
        out_specs=pl.BlockSpec((br, 128), lambda i: (i, 0)),
        out_shape=jax.ShapeDtypeStruct((ep, 128), jnp.float32),
    )(ga_p, gb_p, ewc_p, b1t, w2_blk, b2t)


def _scatter_body(h2_hbm, src_hbm, tgt_hbm, zer_hbm, out_hbm,
                  src_i, tgt_i, h2_v, acc_a, acc_b, sem):
    c = lax.axis_index("c")
    s = lax.axis_index("s")
    wid = c * NS + s
    r0 = s * RPS
    qw = wid // (NW // 4)
    lr0 = (wid % (NW // 4)) * EPW

    pltpu.sync_copy(zer_hbm, acc_a.at[pl.ds(r0, RPS)])
    pltpu.sync_copy(zer_hbm, acc_b.at[pl.ds(r0, RPS)])
    plsc.subcore_barrier()

    def group(g, carry):
        row0 = wid * SRPW + g * SGPC
        rr = lr0 + g * SGE
        pltpu.sync_copy(h2_hbm.at[pl.ds(rr, SGE), pl.ds(EH * qw, EH)], h2_v)
        pltpu.sync_copy(src_hbm.at[pl.ds(row0, SGPC)], src_i)
        pltpu.sync_copy(tgt_hbm.at[pl.ds(row0, SGPC)], tgt_i)

        def burst(j, c2):
            for u in range(5):
                k = j * 5 + u
                sl = h2_v.at[pl.ds(k * SCH, SCH)]
                pltpu.sync_copy(sl, acc_a.at[tgt_i.at[k]], add=True)
                pltpu.sync_copy(sl, acc_b.at[src_i.at[k]], add=True)
            return c2

        lax.fori_loop(0, SGPC // 5, burst, 0)
        return carry

    lax.fori_loop(0, SNG, group, 0)
    plsc.subcore_barrier()

    pltpu.sync_copy(acc_a.at[pl.ds(r0, RPS)],
                    out_hbm.at[2 * c, pl.ds(r0, RPS)])
    pltpu.sync_copy(acc_b.at[pl.ds(r0, RPS)],
                    out_hbm.at[2 * c + 1, pl.ds(r0, RPS)])


def _scatter(h2, src2, tgt2, zer):
    mesh = plsc.VectorSubcoreMesh(core_axis_name="c", subcore_axis_name="s")
    f = pl.kernel(
        _scatter_body,
        out_type=jax.ShapeDtypeStruct((4, N, EH), jnp.float32),
        name="scatter_sc",
        mesh=mesh,
        compiler_params=pltpu.CompilerParams(use_tc_tiling_on_sc=False),
        scratch_types=[
            pltpu.VMEM((SGPC, SCH), jnp.int32),
            pltpu.VMEM((SGPC, SCH), jnp.int32),
            pltpu.VMEM((SGE, EH), jnp.float32),
            pltpu.VMEM_SHARED((N, EH), jnp.float32),
            pltpu.VMEM_SHARED((N, EH), jnp.float32),
            pltpu.SemaphoreType.DMA,
        ],
    )
    return f(h2, src2, tgt2, zer)


def _node_body(p_ref, w3_ref, b3_ref, o_ref):
    p = p_ref[...]
    agg = (p[0] + p[2]) - (p[1] + p[3])
    o_ref[...] = _sig(jnp.dot(agg, w3_ref[...],
                              preferred_element_type=jnp.float32)
                      + b3_ref[...])


def _node_mlp(parts, w3p, b3r):
    bn = 2000
    return pl.pallas_call(
        _node_body,
        grid=(N // bn,),
        in_specs=[
            pl.BlockSpec((4, bn, EH), lambda i: (0, i, 0)),
            pl.BlockSpec((EH, OUT_DIM), lambda i: (0, 0)),
            pl.BlockSpec((1, OUT_DIM), lambda i: (0, 0)),
        ],
        out_specs=pl.BlockSpec((bn, OUT_DIM), lambda i: (i, 0)),
        out_shape=jax.ShapeDtypeStruct((N, OUT_DIM), jnp.float32),
    )(parts, w3p, b3r)


def kernel(x, edge_w, edge_index, W1, b1, W2, b2, W3, b3):
    x2 = x[0].astype(jnp.float32)
    ewT = edge_w[0].T.astype(jnp.float32)
    src = edge_index[0].astype(jnp.int32)
    tgt = edge_index[1].astype(jnp.int32)

    w1a = W1[:IN_DIM]
    w1b = W1[IN_DIM:2 * IN_DIM]
    w1c = W1[2 * IN_DIM:]
    eye4 = jnp.eye(4, dtype=jnp.float32)
    b1t = jnp.tile(b1, 4).reshape(1, 128)
    w2p = jnp.zeros((EH, EH), jnp.float32).at[:, :W2.shape[1]].set(W2)
    b2p = jnp.zeros((EH,), jnp.float32).at[:b2.shape[0]].set(b2)
    w2_blk = jnp.kron(eye4, w2p)
    b2t = jnp.tile(b2p, 4).reshape(1, 128)
    w3p = jnp.zeros((EH, OUT_DIM), jnp.float32).at[:W3.shape[0]].set(W3)
    b3r = b3.reshape(1, OUT_DIM)
    zer = jnp.zeros((RPS, EH), jnp.float32)
    src2 = src.reshape(E // CH, CH)
    tgt2 = tgt.reshape(E // CH, CH)
    src3 = src.reshape(E // SCH, SCH)
    tgt3 = tgt.reshape(E // SCH, SCH)

    xa, xb = _proj(x2, w1a, w1b)
    ewc_p = _ewc(ewT, w1c)
    ga, gb = _gather(xa, xb, src2, tgt2)
    h2_p = _edge_mlp(ga, gb, ewc_p, b1t, w2_blk, b2t)
    parts = _scatter(h2_p, src3, tgt3, zer)
    y = _node_mlp(parts, w3p, b3r)
    return y[None]


# speedup vs baseline: 1.6666x; 1.0298x over previous
import functools

import jax
import jax.numpy as jnp
from jax import lax
from jax.experimental import pallas as pl
from jax.experimental.pallas import tpu as pltpu
from jax.experimental.pallas import tpu_sc as plsc

N = 10000
E = 320000
IN_DIM = 128
OUT_DIM = 128
EH = 32
NC = 2
NS = 16
NW = NC * NS
EPW = E // NW
CH = 40
GPC = 25
GE = CH * GPC
NG = EPW // GE
RPW = EPW // CH
SCH = 80
SGPC = 25
SGE = SCH * SGPC
SNG = EPW // SGE
SRPW = EPW // SCH
RPS = N // NS


def _sig(t):
    return 1.0 / (1.0 + jnp.exp(-t))


def _proj_body(x_ref, wa_ref, wb_ref, xa_ref, xb_ref):
    xv = x_ref[...]
    xa_ref[...] = jnp.dot(xv, wa_ref[...],
                          preferred_element_type=jnp.float32)
    xb_ref[...] = jnp.dot(xv, wb_ref[...],
                          preferred_element_type=jnp.float32)


def _proj(x2, w1a, w1b):
    bn = 2000
    return pl.pallas_call(
        _proj_body,
        grid=(N // bn,),
        in_specs=[
            pl.BlockSpec((bn, IN_DIM), lambda i: (i, 0)),
            pl.BlockSpec((IN_DIM, EH), lambda i: (0, 0)),
            pl.BlockSpec((IN_DIM, EH), lambda i: (0, 0)),
        ],
        out_specs=[
            pl.BlockSpec((bn, EH), lambda i: (i, 0)),
            pl.BlockSpec((bn, EH), lambda i: (i, 0)),
        ],
        out_shape=[
            jax.ShapeDtypeStruct((N, EH), jnp.float32),
            jax.ShapeDtypeStruct((N, EH), jnp.float32),
        ],
    )(x2, w1a, w1b)


def _ewc_body(ew0_ref, ew1_ref, ew2_ref, ew3_ref, w1c_ref, o_ref):
    dn = (((0,), (0,)), ((), ()))
    w1c = w1c_ref[...]
    o_ref[...] = jnp.concatenate(
        [lax.dot_general(r[...], w1c, dn, preferred_element_type=jnp.float32)
         for r in (ew0_ref, ew1_ref, ew2_ref, ew3_ref)],
        axis=1)


def _ewc(ewT, w1c):
    br = 3200
    ep = E // 4
    nb = ep // br
    return pl.pallas_call(
        _ewc_body,
        grid=(nb,),
        in_specs=[
            pl.BlockSpec((3, br), lambda i: (0, i)),
            pl.BlockSpec((3, br), lambda i: (0, i + nb)),
            pl.BlockSpec((3, br), lambda i: (0, i + 2 * nb)),
            pl.BlockSpec((3, br), lambda i: (0, i + 3 * nb)),
            pl.BlockSpec((3, EH), lambda i: (0, 0)),
        ],
        out_specs=pl.BlockSpec((br, 128), lambda i: (i, 0)),
        out_shape=jax.ShapeDtypeStruct((ep, 128), jnp.float32),
    )(ewT, ewT, ewT, ewT, w1c)


def _gather_body(xa_hbm, xb_hbm, src_hbm, tgt_hbm, ga_hbm, gb_hbm,
                 src_i, tgt_i, ga_v, gb_v, sem, wsem):
    c = lax.axis_index("c")
    s = lax.axis_index("s")
    wid = c * NS + s

    qw = wid // (NW // 4)
    lr0 = (wid % (NW // 4)) * EPW
    sb = GPC // 5

    def drain_sub(buf_v, j):
        pltpu.make_async_copy(xa_hbm.at[pl.ds(0, 5 * CH)],
                              buf_v.at[pl.ds(j * 5 * CH, 5 * CH)],
                              sem).wait()

    def drain_writes():
        pltpu.make_async_copy(xa_hbm.at[pl.ds(0, GE)], ga_v, wsem).wait()
        pltpu.make_async_copy(xa_hbm.at[pl.ds(0, GE)], gb_v, wsem).wait()

    def group(g, carry):
        row0 = wid * RPW + g * GPC
        rr = lr0 + g * GE
        @pl.when(g > 0)
        def _():
            drain_writes()
        pltpu.sync_copy(src_hbm.at[pl.ds(row0, GPC)], src_i)
        pltpu.sync_copy(tgt_hbm.at[pl.ds(row0, GPC)], tgt_i)

        def burst(j, c2):
            for u in range(5):
                k = j * 5 + u
                pltpu.async_copy(xa_hbm.at[src_i.at[k]],
                                 ga_v.at[pl.ds(k * CH, CH)], sem)
                pltpu.async_copy(xb_hbm.at[tgt_i.at[k]],
                                 gb_v.at[pl.ds(k * CH, CH)], sem)
            @pl.when(j > 0)
            def _():
                drain_sub(ga_v, j - 1)
                drain_sub(gb_v, j - 1)
            return c2

        lax.fori_loop(0, sb, burst, 0)
        drain_sub(ga_v, sb - 1)
        drain_sub(gb_v, sb - 1)
        pltpu.async_copy(ga_v, ga_hbm.at[pl.ds(rr, GE), pl.ds(EH * qw, EH)],
                         wsem)
        pltpu.async_copy(gb_v, gb_hbm.at[pl.ds(rr, GE), pl.ds(EH * qw, EH)],
                         wsem)
        return carry

    lax.fori_loop(0, NG, group, 0)
    drain_writes()


def _gather(xa, xb, src2, tgt2):
    mesh = plsc.VectorSubcoreMesh(core_axis_name="c", subcore_axis_name="s")
    f = pl.kernel(
        _gather_body,
        out_type=[
            jax.ShapeDtypeStruct((E // 4, 128), jnp.float32),
            jax.ShapeDtypeStruct((E // 4, 128), jnp.float32),
        ],
        mesh=mesh,
        compiler_params=pltpu.CompilerParams(use_tc_tiling_on_sc=False),
        scratch_types=[
            pltpu.VMEM((GPC, CH), jnp.int32),
            pltpu.VMEM((GPC, CH), jnp.int32),
            pltpu.VMEM((GE, EH), jnp.float32),
            pltpu.VMEM((GE, EH), jnp.float32),
            pltpu.SemaphoreType.DMA,
            pltpu.SemaphoreType.DMA,
        ],
    )
    return f(xa, xb, src2, tgt2)


def _edge_body(ga_ref, gb_ref, ewc_ref, b1_ref, w2_ref, b2_ref, o_ref):
    pre1 = ga_ref[...] + gb_ref[...] + ewc_ref[...] + b1_ref[...]
    h1 = _sig(pre1)
    o_ref[...] = _sig(jnp.dot(h1, w2_ref[...],
                              preferred_element_type=jnp.float32)
                      + b2_ref[...])


def _edge_mlp(ga_p, gb_p, ewc_p, b1t, w2_blk, b2t):
    br = 4000
    ep = E // 4
    nb = ep // br
    return pl.pallas_call(
        _edge_body,
        grid=(nb,),
        in_specs=[
            pl.BlockSpec((br, 128), lambda i: (i, 0)),
            pl.BlockSpec((br, 128), lambda i: (i, 0)),
            pl.BlockSpec((br, 128), lambda i: (i, 0)),
            pl.BlockSpec((1, 128), lambda i: (0, 0)),
            pl.BlockSpec((128, 128), lambda i: (0, 0)),
            pl.BlockSpec((1, 128), lambda i: (0, 0)),
        ],
        out_specs=pl.BlockSpec((br, 128), lambda i: (i, 0)),
        out_shape=jax.ShapeDtypeStruct((ep, 128), jnp.float32),
    )(ga_p, gb_p, ewc_p, b1t, w2_blk, b2t)


def _scatter_body(h2_hbm, src_hbm, tgt_hbm, zer_hbm, out_hbm,
                  src_i, tgt_i, h2_v, acc_a, acc_b, sem):
    c = lax.axis_index("c")
    s = lax.axis_index("s")
    wid = c * NS + s
    r0 = s * RPS
    qw = wid // (NW // 4)
    lr0 = (wid % (NW // 4)) * EPW

    pltpu.sync_copy(zer_hbm, acc_a.at[pl.ds(r0, RPS)])
    pltpu.sync_copy(zer_hbm, acc_b.at[pl.ds(r0, RPS)])
    plsc.subcore_barrier()

    def group(g, carry):
        row0 = wid * SRPW + g * SGPC
        rr = lr0 + g * SGE
        pltpu.sync_copy(h2_hbm.at[pl.ds(rr, SGE), pl.ds(EH * qw, EH)], h2_v)
        pltpu.sync_copy(src_hbm.at[pl.ds(row0, SGPC)], src_i)
        pltpu.sync_copy(tgt_hbm.at[pl.ds(row0, SGPC)], tgt_i)

        def burst(j, c2):
            for u in range(5):
                k = j * 5 + u
                sl = h2_v.at[pl.ds(k * SCH, SCH)]
                pltpu.sync_copy(sl, acc_a.at[tgt_i.at[k]], add=True)
                pltpu.sync_copy(sl, acc_b.at[src_i.at[k]], add=True)
            return c2

        lax.fori_loop(0, SGPC // 5, burst, 0)
        return carry

    lax.fori_loop(0, SNG, group, 0)
    plsc.subcore_barrier()

    pltpu.sync_copy(acc_a.at[pl.ds(r0, RPS)],
                    out_hbm.at[2 * c, pl.ds(r0, RPS)])
    pltpu.sync_copy(acc_b.at[pl.ds(r0, RPS)],
                    out_hbm.at[2 * c + 1, pl.ds(r0, RPS)])


def _scatter(h2, src2, tgt2, zer):
    mesh = plsc.VectorSubcoreMesh(core_axis_name="c", subcore_axis_name="s")
    f = pl.kernel(
        _scatter_body,
        out_type=jax.ShapeDtypeStruct((4, N, EH), jnp.float32),
        name="scatter_sc",
        mesh=mesh,
        compiler_params=pltpu.CompilerParams(use_tc_tiling_on_sc=False),
        scratch_types=[
            pltpu.VMEM((SGPC, SCH), jnp.int32),
            pltpu.VMEM((SGPC, SCH), jnp.int32),
            pltpu.VMEM((SGE, EH), jnp.float32),
            pltpu.VMEM_SHARED((N, EH), jnp.float32),
            pltpu.VMEM_SHARED((N, EH), jnp.float32),
            pltpu.SemaphoreType.DMA,
        ],
    )
    return f(h2, src2, tgt2, zer)


def _node_body(p_ref, w3_ref, b3_ref, o_ref):
    p = p_ref[...]
    aggp = (p[0] + p[2]) - (p[1] + p[3])
    w3 = w3_ref[...]
    b3 = b3_ref[...]
    ys = [_sig(jnp.dot(aggp[:, q * EH:(q + 1) * EH], w3,
                       preferred_element_type=jnp.float32) + b3)
          for q in range(4)]
    o_ref[...] = jnp.concatenate(ys, axis=1)


def _node_mlp(parts_p, w3p, b3r):
    bn4 = N // 4
    return pl.pallas_call(
        _node_body,
        grid=(1,),
        in_specs=[
            pl.BlockSpec((4, bn4, 128), lambda i: (0, i, 0)),
            pl.BlockSpec((EH, OUT_DIM), lambda i: (0, 0)),
            pl.BlockSpec((1, OUT_DIM), lambda i: (0, 0)),
        ],
        out_specs=pl.BlockSpec((bn4, 4 * OUT_DIM), lambda i: (i, 0)),
        out_shape=jax.ShapeDtypeStruct((N // 4, 4 * OUT_DIM), jnp.float32),
    )(parts_p, w3p, b3r)


def kernel(x, edge_w, edge_index, W1, b1, W2, b2, W3, b3):
    x2 = x[0].astype(jnp.float32)
    ewT = edge_w[0].T.astype(jnp.float32)
    src = edge_index[0].astype(jnp.int32)
    tgt = edge_index[1].astype(jnp.int32)

    w1a = W1[:IN_DIM]
    w1b = W1[IN_DIM:2 * IN_DIM]
    w1c = W1[2 * IN_DIM:]
    eye4 = jnp.eye(4, dtype=jnp.float32)
    b1t = jnp.tile(b1, 4).reshape(1, 128)
    w2p = jnp.zeros((EH, EH), jnp.float32).at[:, :W2.shape[1]].set(W2)
    b2p = jnp.zeros((EH,), jnp.float32).at[:b2.shape[0]].set(b2)
    w2_blk = jnp.kron(eye4, w2p)
    b2t = jnp.tile(b2p, 4).reshape(1, 128)
    w3p = jnp.zeros((EH, OUT_DIM), jnp.float32).at[:W3.shape[0]].set(W3)
    b3r = b3.reshape(1, OUT_DIM)
    zer = jnp.zeros((RPS, EH), jnp.float32)
    src2 = src.reshape(E // CH, CH)
    tgt2 = tgt.reshape(E // CH, CH)
    src3 = src.reshape(E // SCH, SCH)
    tgt3 = tgt.reshape(E // SCH, SCH)

    xa, xb = _proj(x2, w1a, w1b)
    ewc_p = _ewc(ewT, w1c)
    ga, gb = _gather(xa, xb, src2, tgt2)
    h2_p = _edge_mlp(ga, gb, ewc_p, b1t, w2_blk, b2t)
    parts = _scatter(h2_p, src3, tgt3, zer)
    y_p = _node_mlp(parts.reshape(4, N // 4, 128), w3p, b3r)
    return y_p.reshape(N, OUT_DIM)[None]
